# baseline (device time: 43436 ns/iter reference)
import os

import jax
import jax.numpy as jnp
from jax import lax
from jax.experimental import pallas as pl
from jax.experimental.pallas import tpu as pltpu

_BENCH = os.environ.get("BENCH", "")

N_DEV = 4
B, SQ, D = 4, 256, 1024
DH = 128
SCALE = 0.08838834764831843
ROWS = B * SQ
U = 128


def kernel(x, Wq, Wo, Wk, Wv):
    n_heads = Wq.shape[1] // DH
    x2 = x.reshape(ROWS, D)

    def body(x_ref, wq_ref, wo_ref, wk_ref, wv_ref, out_ref,
             sbuf_ref, rbuf_ref, x16_ref, send_sems, recv_sems):
        my = lax.axis_index("i")
        a_half = (my ^ (my >> 1)) & 1
        b_half = (my >> 1) & 1
        qa_sub = b_half
        qb_sub = my & 1
        p1 = my ^ 1
        p2 = 3 - my

        barrier_sem = pltpu.get_barrier_semaphore()
        for nbr in (p1, p2):
            pl.semaphore_signal(
                barrier_sem, inc=1,
                device_id=(nbr,), device_id_type=pl.DeviceIdType.MESH,
            )
        pl.semaphore_wait(barrier_sem, 2)

        wq16 = wq_ref[...].astype(jnp.bfloat16)
        wk16 = wk_ref[...].astype(jnp.bfloat16)
        wv16 = wv_ref[...].astype(jnp.bfloat16)
        wo16 = wo_ref[...].astype(jnp.bfloat16)
        x16_ref[...] = x_ref[...].astype(jnp.bfloat16)

        def attend(qm, km, vm):
            outs = []
            for h in range(n_heads):
                c0 = h * DH
                qh = qm[:, c0:c0 + DH]
                kh = km[:, c0:c0 + DH]
                vh = vm[:, c0:c0 + DH]
                s = lax.dot_general(
                    qh, kh, (((1,), (1,)), ((), ())),
                    preferred_element_type=jnp.float32,
                ) * SCALE
                m = jnp.max(s, axis=-1, keepdims=True)
                p = jnp.exp(s - m)
                l = jnp.sum(p, axis=-1, keepdims=True)
                pv = jnp.dot(
                    p.astype(jnp.bfloat16), vh,
                    preferred_element_type=jnp.float32,
                )
                outs.append(pv / l)
            return jnp.concatenate(outs, axis=1).astype(jnp.bfloat16)

        def compute_batch(bi, send_slot=None):
            r = pl.ds(bi * SQ, SQ)
            xb = x16_ref[r, :]
            qm = jnp.dot(
                xb, wq16, preferred_element_type=jnp.float32
            ).astype(jnp.bfloat16)
            km = jnp.dot(
                xb, wk16, preferred_element_type=jnp.float32
            ).astype(jnp.bfloat16)
            vm = jnp.dot(
                xb, wv16, preferred_element_type=jnp.float32
            ).astype(jnp.bfloat16)
            attn_b = attend(qm, km, vm)
            out_b = jnp.dot(
                attn_b, wo16, preferred_element_type=jnp.float32
            )
            out_ref[r, :] = out_b
            if send_slot is not None:
                sbuf_ref[send_slot, :, :] = out_b.astype(jnp.bfloat16)

        def compute_half(bi, half, kv=None):
            if kv is None:
                xbf = x16_ref[pl.ds(bi * SQ, SQ), :]
                km = jnp.dot(
                    xbf, wk16, preferred_element_type=jnp.float32
                ).astype(jnp.bfloat16)
                vm = jnp.dot(
                    xbf, wv16, preferred_element_type=jnp.float32
                ).astype(jnp.bfloat16)
            else:
                km, vm = kv
            r = pl.ds(bi * SQ + half * U, U)
            xh = x16_ref[r, :]
            qm = jnp.dot(
                xh, wq16, preferred_element_type=jnp.float32
            ).astype(jnp.bfloat16)
            attn_h = attend(qm, km, vm)
            out_ref[r, :] = jnp.dot(
                attn_h, wo16, preferred_element_type=jnp.float32
            )
            return (km, vm)

        def exch(slot, n_u, partner, sem):
            rdma = pltpu.make_async_remote_copy(
                src_ref=sbuf_ref.at[slot, pl.ds(0, n_u * U), :],
                dst_ref=rbuf_ref.at[slot, pl.ds(0, n_u * U), :],
                send_sem=send_sems.at[sem],
                recv_sem=recv_sems.at[sem],
                device_id=(partner,),
                device_id_type=pl.DeviceIdType.MESH,
            )
            rdma.start()
            return rdma

        def stage(slot, src_u, n_u):
            rows = pl.ds(src_u * U, n_u * U)
            sbuf_ref[slot, pl.ds(0, n_u * U), :] = (
                out_ref[rows, :].astype(jnp.bfloat16)
            )

        def accum(slot, dst_u, n_u):
            rows = pl.ds(dst_u * U, n_u * U)
            out_ref[rows, :] = out_ref[rows, :] + (
                rbuf_ref[slot, pl.ds(0, n_u * U), :].astype(jnp.float32)
            )

        def store(slot, dst_u, n_u):
            rows = pl.ds(dst_u * U, n_u * U)
            out_ref[rows, :] = (
                rbuf_ref[slot, pl.ds(0, n_u * U), :].astype(jnp.float32)
            )

        if _BENCH == "compute":
            for bi in range(B):
                compute_batch(bi)
            return
        if _BENCH == "matmul":
            f8 = jnp.float8_e4m3fn
            wq8 = (wq_ref[...] * 50.0).astype(f8)
            wk8 = (wk_ref[...] * 50.0).astype(f8)
            wv8 = (wv_ref[...] * 50.0).astype(f8)
            wo8 = (wo_ref[...] * 50.0).astype(f8)
            for bi in range(B):
                r = pl.ds(bi * SQ, SQ)
                xb = x_ref[r, :].astype(f8)
                qm = jnp.dot(xb, wq8, preferred_element_type=jnp.float32)
                km = jnp.dot(xb, wk8, preferred_element_type=jnp.float32)
                vm = jnp.dot(xb, wv8, preferred_element_type=jnp.float32)
                acc = ((qm + km + vm) * 0.02).astype(f8)
                out_ref[r, :] = jnp.dot(
                    acc, wo8, preferred_element_type=jnp.float32
                )
            return
        def accum_stage(rslot, dst_u, sslot):
            rows = pl.ds(dst_u * U, 2 * U)
            val = out_ref[rows, :] + (
                rbuf_ref[rslot, :, :].astype(jnp.float32)
            )
            out_ref[rows, :] = val
            sbuf_ref[sslot, :, :] = val.astype(jnp.bfloat16)

        def accum_stage_u(rslot, rsub, dst_u, sslot):
            rows = pl.ds(dst_u * U, U)
            val = out_ref[rows, :] + (
                rbuf_ref[rslot, pl.ds(rsub * U, U), :].astype(jnp.float32)
            )
            out_ref[rows, :] = val
            sbuf_ref[sslot, pl.ds(rsub * U, U), :] = val.astype(jnp.bfloat16)

        def exch_u(slot, sub, partner, sem):
            rdma = pltpu.make_async_remote_copy(
                src_ref=sbuf_ref.at[slot, pl.ds(sub * U, U), :],
                dst_ref=rbuf_ref.at[slot, pl.ds(sub * U, U), :],
                send_sem=send_sems.at[sem],
                recv_sem=recv_sems.at[sem],
                device_id=(partner,),
                device_id_type=pl.DeviceIdType.MESH,
            )
            rdma.start()
            return rdma

        ua = 2 * a_half
        ub = 4 + 2 * b_half

        compute_batch(1 - a_half, send_slot=0)
        ra = exch(0, 2, p1, 0)
        compute_batch(3 - b_half, send_slot=1)
        rb = exch(1, 2, p2, 1)
        compute_batch(a_half)
        ra.wait()
        accum_stage(0, ua, 2)
        ra = exch(2, 2, p2, 2)

        kv = compute_half(2 + b_half, 0)
        rb.wait()
        accum_stage_u(1, 0, ub, 3)
        rb2a = exch_u(3, 0, p1, 3)
        compute_half(2 + b_half, 1, kv)
        accum_stage_u(1, 1, ub + 1, 3)
        rb2b = exch_u(3, 1, p1, 6)

        ra.wait()
        accum_stage(2, ua, 4)
        ra = exch(4, 2, p1, 4)
        rb2a.wait()
        accum_stage_u(3, 0, ub, 5)
        rb3a = exch_u(5, 0, p2, 5)
        rb2b.wait()
        accum_stage_u(3, 1, ub + 1, 5)
        rb3b = exch_u(5, 1, p2, 7)

        ra.wait()
        store(4, 2 * (1 - a_half), 2)
        rb3a.wait()
        rb3b.wait()
        store(5, 4 + 2 * (1 - b_half), 2)

    out2 = pl.pallas_call(
        body,
        out_shape=jax.ShapeDtypeStruct((ROWS, D), jnp.float32),
        in_specs=[pl.BlockSpec(memory_space=pltpu.VMEM)] * 5,
        out_specs=pl.BlockSpec(memory_space=pltpu.VMEM),
        scratch_shapes=[
            pltpu.VMEM((8, 2 * U, D), jnp.bfloat16),
            pltpu.VMEM((8, 2 * U, D), jnp.bfloat16),
            pltpu.VMEM((ROWS, D), jnp.bfloat16),
            pltpu.SemaphoreType.DMA((8,)),
            pltpu.SemaphoreType.DMA((8,)),
        ],
        compiler_params=pltpu.CompilerParams(collective_id=0),
    )(x2, Wq, Wo, Wk, Wv)
    return out2.reshape(B, SQ, D)
